# Initial kernel scaffold; baseline (speedup 1.0000x reference)
import jax
import jax.numpy as jnp
from jax.experimental import pallas as pl


def kernel(pred, target, weight):
    raise NotImplementedError("write your pallas kernel here")


# speedup vs baseline: 2.6334x; 2.6334x over previous
import functools

import jax
import jax.numpy as jnp
from jax import lax
from jax.experimental import pallas as pl
from jax.experimental.pallas import tpu as pltpu
from jax.experimental.pallas import tpu_sc as plsc

_BINS = 10
_L = 16
_NC = 2
_NS = 16
_NW = _NC * _NS

_LOG1P_C = (
    5.23940263e-09, 9.99998911e-01, -4.99962245e-01, 3.32818425e-01,
    -2.46356606e-01, 1.84688485e-01, -1.25266614e-01, 6.65124792e-02,
    -2.30382799e-02, 3.75262421e-03,
)


@functools.lru_cache(maxsize=None)
def _make_hist_kernel(total: int, chunk: int):
    per_w = total // _NW
    assert total % _NW == 0
    assert per_w % chunk == 0
    assert chunk % _L == 0 and chunk % 8 == 0
    n_chunks = per_w // chunk
    n_vregs = chunk // _L

    mesh = plsc.VectorSubcoreMesh(core_axis_name="c", subcore_axis_name="s")

    @functools.partial(
        pl.kernel,
        out_type=jax.ShapeDtypeStruct((_NW, 2 * _L), jnp.float32),
        mesh=mesh,
        scratch_types=[
            pltpu.VMEM((chunk,), jnp.float32),
            pltpu.VMEM((chunk,), jnp.float32),
            pltpu.VMEM((chunk,), jnp.float32),
            pltpu.VMEM((chunk,), jnp.float32),
            pltpu.VMEM((chunk,), jnp.float32),
            pltpu.VMEM((chunk,), jnp.float32),
            pltpu.VMEM((_L * _L,), jnp.float32),
            pltpu.VMEM((_L * _L,), jnp.float32),
            pltpu.VMEM((2 * _L,), jnp.float32),
            pltpu.SemaphoreType.DMA,
            pltpu.SemaphoreType.DMA,
        ],
        compiler_params=pltpu.CompilerParams(needs_layout_passes=False),
    )
    def hist_kernel(p_hbm, t_hbm, w_hbm, out_hbm,
                    pbuf0, pbuf1, tbuf0, tbuf1, wbuf0, wbuf1,
                    accw, accc, res, sem0, sem1):
        wid = lax.axis_index("s") * _NC + lax.axis_index("c")
        base = wid * per_w
        sems = (sem0, sem1)
        pbufs = (pbuf0, pbuf1)
        tbufs = (tbuf0, tbuf1)
        wbufs = (wbuf0, wbuf1)

        zeros = jnp.zeros((_L,), jnp.float32)
        for i in range(_L):
            accw[pl.ds(i * _L, _L)] = zeros
            accc[pl.ds(i * _L, _L)] = zeros

        def issue(ci):
            b = ci % 2
            off = base + ci * chunk
            return (
                pltpu.async_copy(p_hbm.at[pl.ds(off, chunk)], pbufs[b], sems[b]),
                pltpu.async_copy(t_hbm.at[pl.ds(off, chunk)], tbufs[b], sems[b]),
                pltpu.async_copy(w_hbm.at[pl.ds(off, chunk)], wbufs[b], sems[b]),
            )

        lanebase = lax.iota(jnp.int32, _L) * _L
        ones = jnp.ones((_L,), jnp.float32)

        pending = {0: issue(0)}
        for ci in range(n_chunks):
            if ci + 1 < n_chunks:
                pending[ci + 1] = issue(ci + 1)
            for d in pending.pop(ci):
                d.wait()
            b = ci % 2
            pb, tb, wb = pbufs[b], tbufs[b], wbufs[b]

            def body(j, carry):
                off = j * _L
                p = pb[pl.ds(off, _L)]
                t = tb[pl.ds(off, _L)]
                w = wb[pl.ds(off, _L)]
                e = jnp.exp(-p)
                s = 1.0 / (1.0 + e)
                g = jnp.abs(s - t)
                valid = w > 0.0
                bidx = jnp.minimum(g * 10.0, 9.0).astype(jnp.int32)
                bidx = jnp.where(valid, bidx, 15)
                idx = lanebase + bidx
                u = jnp.minimum(e, 1.0 / e)
                acc = jnp.full((_L,), _LOG1P_C[-1], dtype=jnp.float32)
                for c in _LOG1P_C[-2::-1]:
                    acc = acc * u + c
                bce = jnp.maximum(p, 0.0) - p * t + acc
                plsc.addupdate_scatter(accw, [idx], bce * w)
                plsc.addupdate_scatter(accc, [idx], ones)
                return carry

            lax.fori_loop(0, n_vregs, body, 0)

        wv = accw[pl.ds(0, _L)]
        cv = accc[pl.ds(0, _L)]
        for l in range(1, _L):
            wv = wv + accw[pl.ds(l * _L, _L)]
            cv = cv + accc[pl.ds(l * _L, _L)]
        res[pl.ds(0, _L)] = wv
        res[pl.ds(_L, _L)] = cv
        pltpu.sync_copy(res, out_hbm.at[wid])

    return hist_kernel


def kernel(pred, target, weight):
    total = pred.size
    pf = pred.reshape(-1)
    tf = target.reshape(-1)
    wf = weight.reshape(-1)
    parts = _make_hist_kernel(total, 10000)(pf, tf, wf)
    sums = jnp.sum(parts, axis=0)
    w_b = sums[:_BINS]
    c_b = sums[_L:_L + _BINS]
    nne = jnp.sum((c_b > 0).astype(jnp.float32))
    loss = jnp.sum(jnp.where(c_b > 0, w_b / jnp.maximum(c_b, 1.0), 0.0))
    return loss / jnp.maximum(nne, 1.0)
